# baseline (speedup 1.0000x reference)
import jax
import jax.numpy as jnp
from jax import lax
from jax.experimental import pallas as pl
from jax.experimental.pallas import tpu as pltpu
from jax.experimental.pallas import tpu_sc as plsc

VOCAB = 1000000
EMBED = 64
BATCH = 1024
CTX = 5

_NC = 2
_NS = 16
_NW = _NC * _NS

_TGT_PER_W = BATCH // _NW
_CTX_PER_W = BATCH * CTX // _NW
_CTX_HALF = _CTX_PER_W // 2


def _gather_body(wt_hbm, wc_hbm, tidx_hbm, cidx_hbm, out_t_hbm, out_c_hbm,
                 tidx_v, cidx_v0, cidx_v1, trows_v, crows_v0, crows_v1,
                 sem_t, sem_c):
    wid = lax.axis_index("s") * _NC + lax.axis_index("c")
    base_t = wid * _TGT_PER_W
    base_c = wid * _CTX_PER_W
    pltpu.sync_copy(tidx_hbm.at[pl.ds(base_t, _TGT_PER_W)], tidx_v)
    pltpu.sync_copy(cidx_hbm.at[pl.ds(base_c, _CTX_HALF)], cidx_v0)
    pltpu.sync_copy(cidx_hbm.at[pl.ds(base_c + _CTX_HALF, _CTX_HALF)], cidx_v1)
    ct = pltpu.async_copy(wt_hbm.at[tidx_v], trows_v, sem_t)
    c0 = pltpu.async_copy(wc_hbm.at[cidx_v0], crows_v0, sem_c)
    c1 = pltpu.async_copy(wc_hbm.at[cidx_v1], crows_v1, sem_c)
    ct.wait()
    c0.wait()
    c1.wait()
    pltpu.sync_copy(trows_v, out_t_hbm.at[pl.ds(base_t, _TGT_PER_W)])
    pltpu.sync_copy(crows_v0, out_c_hbm.at[pl.ds(base_c, _CTX_HALF)])
    pltpu.sync_copy(crows_v1, out_c_hbm.at[pl.ds(base_c + _CTX_HALF, _CTX_HALF)])


_gather = pl.kernel(
    _gather_body,
    out_type=(
        jax.ShapeDtypeStruct((BATCH, EMBED), jnp.float32),
        jax.ShapeDtypeStruct((BATCH * CTX, EMBED), jnp.float32),
    ),
    mesh=plsc.VectorSubcoreMesh(core_axis_name="c", subcore_axis_name="s"),
    compiler_params=pltpu.CompilerParams(use_tc_tiling_on_sc=False),
    scratch_types=[
        pltpu.VMEM((_TGT_PER_W,), jnp.int32),
        pltpu.VMEM((_CTX_HALF,), jnp.int32),
        pltpu.VMEM((_CTX_HALF,), jnp.int32),
        pltpu.VMEM((_TGT_PER_W, EMBED), jnp.float32),
        pltpu.VMEM((_CTX_HALF, EMBED), jnp.float32),
        pltpu.VMEM((_CTX_HALF, EMBED), jnp.float32),
        pltpu.SemaphoreType.DMA,
        pltpu.SemaphoreType.DMA,
    ],
)


def _mm_body(a_ref, b_ref, o_ref):
    o_ref[...] = lax.dot_general(
        a_ref[...], b_ref[...],
        dimension_numbers=(((1,), (1,)), ((), ())),
        preferred_element_type=jnp.float32,
    )


_BN = 640


def kernel(target, context, W_target, W_context):
    tidx = jnp.asarray(target, jnp.int32).reshape(BATCH)
    cidx = jnp.asarray(context, jnp.int32).reshape(BATCH * CTX)
    emb_t, emb_c = _gather(W_target, W_context, tidx, cidx)
    out2d = pl.pallas_call(
        _mm_body,
        grid=(BATCH * CTX // _BN,),
        in_specs=[
            pl.BlockSpec((BATCH, EMBED), lambda j: (0, 0)),
            pl.BlockSpec((_BN, EMBED), lambda j: (j, 0)),
        ],
        out_specs=pl.BlockSpec((BATCH, _BN), lambda j: (0, j)),
        out_shape=jax.ShapeDtypeStruct((BATCH, BATCH * CTX), jnp.float32),
    )(emb_t, emb_c)
    return out2d.reshape(BATCH, BATCH, CTX)


# speedup vs baseline: 1.5276x; 1.5276x over previous
import jax
import jax.numpy as jnp
from jax import lax
from jax.experimental import pallas as pl
from jax.experimental.pallas import tpu as pltpu
from jax.experimental.pallas import tpu_sc as plsc

VOCAB = 1000000
EMBED = 64
BATCH = 1024
CTX = 5

_NC = 2
_NS = 16
_NW = _NC * _NS

_TGT_PER_W = BATCH // _NW
_CTX_PER_W = BATCH * CTX // _NW


def _gather_body(wt_hbm, wc_hbm, tidx_hbm, cidx_hbm, out_t_hbm, out_c_hbm,
                 tidx_v, cidx_v, trows_v, crows_v,
                 sem_t, sem_c):
    wid = lax.axis_index("s") * _NC + lax.axis_index("c")
    base_t = wid * _TGT_PER_W
    base_c = wid * _CTX_PER_W
    pltpu.sync_copy(tidx_hbm.at[pl.ds(base_t, _TGT_PER_W)], tidx_v)
    pltpu.sync_copy(cidx_hbm.at[pl.ds(base_c, _CTX_PER_W)], cidx_v)

    def fire_t(k, _):
        vec = tidx_v[pl.ds(k * 16, 16)]
        for j in range(16):
            pltpu.async_copy(wt_hbm.at[pl.ds(vec[j], 1), :],
                             trows_v.at[pl.ds(k * 16 + j, 1), :], sem_t)
        return 0

    def fire_c(k, _):
        vec = cidx_v[pl.ds(k * 16, 16)]
        for j in range(16):
            pltpu.async_copy(wc_hbm.at[pl.ds(vec[j], 1), :],
                             crows_v.at[pl.ds(k * 16 + j, 1), :], sem_c)
        return 0

    lax.fori_loop(0, _TGT_PER_W // 16, fire_t, 0)
    lax.fori_loop(0, _CTX_PER_W // 16, fire_c, 0)
    pltpu.make_async_copy(wt_hbm.at[pl.ds(0, _TGT_PER_W), :], trows_v,
                          sem_t).wait()
    pltpu.make_async_copy(wc_hbm.at[pl.ds(0, _CTX_PER_W), :], crows_v,
                          sem_c).wait()
    pltpu.sync_copy(trows_v, out_t_hbm.at[pl.ds(base_t, _TGT_PER_W)])
    pltpu.sync_copy(crows_v, out_c_hbm.at[pl.ds(base_c, _CTX_PER_W)])


_gather = pl.kernel(
    _gather_body,
    out_type=(
        jax.ShapeDtypeStruct((BATCH, EMBED), jnp.float32),
        jax.ShapeDtypeStruct((BATCH * CTX, EMBED), jnp.float32),
    ),
    mesh=plsc.VectorSubcoreMesh(core_axis_name="c", subcore_axis_name="s"),
    scratch_types=[
        pltpu.VMEM((_TGT_PER_W,), jnp.int32),
        pltpu.VMEM((_CTX_PER_W,), jnp.int32),
        pltpu.VMEM((_TGT_PER_W, EMBED), jnp.float32),
        pltpu.VMEM((_CTX_PER_W, EMBED), jnp.float32),
        pltpu.SemaphoreType.DMA,
        pltpu.SemaphoreType.DMA,
    ],
)


def _mm_body(a_ref, b_ref, o_ref):
    o_ref[...] = lax.dot_general(
        a_ref[...], b_ref[...],
        dimension_numbers=(((1,), (1,)), ((), ())),
        preferred_element_type=jnp.float32,
    )


_BN = 640


def kernel(target, context, W_target, W_context):
    tidx = jnp.asarray(target, jnp.int32).reshape(BATCH)
    cidx = jnp.asarray(context, jnp.int32).reshape(BATCH * CTX)
    emb_t, emb_c = _gather(W_target, W_context, tidx, cidx)
    out2d = pl.pallas_call(
        _mm_body,
        grid=(BATCH * CTX // _BN,),
        in_specs=[
            pl.BlockSpec((BATCH, EMBED), lambda j: (0, 0)),
            pl.BlockSpec((_BN, EMBED), lambda j: (j, 0)),
        ],
        out_specs=pl.BlockSpec((BATCH, _BN), lambda j: (0, j)),
        out_shape=jax.ShapeDtypeStruct((BATCH, BATCH * CTX), jnp.float32),
    )(emb_t, emb_c)
    return out2d.reshape(BATCH, BATCH, CTX)


# speedup vs baseline: 2.2543x; 1.4757x over previous
import jax
import jax.numpy as jnp
from jax import lax
from jax.experimental import pallas as pl
from jax.experimental.pallas import tpu as pltpu
from jax.experimental.pallas import tpu_sc as plsc

VOCAB = 1000000
EMBED = 64
BATCH = 1024
CTX = 5

_NC = 2
_NS = 16
_NW = _NC * _NS

_TGT_PER_W = BATCH // _NW
_CTX_PER_W = BATCH * CTX // _NW


def _gather_body(wt_hbm, wc_hbm, tidx_hbm, cidx_hbm, out_t_hbm, out_c_hbm,
                 tidx_v, cidx_v, trows_v, crows_v,
                 sem_t, sem_c):
    wid = lax.axis_index("s") * _NC + lax.axis_index("c")
    base_t = wid * _TGT_PER_W
    base_c = wid * _CTX_PER_W
    pltpu.sync_copy(tidx_hbm.at[pl.ds(base_t, _TGT_PER_W)], tidx_v)
    pltpu.sync_copy(cidx_hbm.at[pl.ds(base_c, _CTX_PER_W)], cidx_v)

    def fire_t(k, _):
        vec = tidx_v[pl.ds(k * 16, 16)]
        for j in range(16):
            r = vec[j]
            pltpu.async_copy(wt_hbm.at[r >> 4, pl.ds(r & 15, 1), :],
                             trows_v.at[pl.ds(k * 16 + j, 1), :], sem_t)
        return 0

    def fire_c(k, _):
        vec = cidx_v[pl.ds(k * 16, 16)]
        for j in range(16):
            r = vec[j]
            pltpu.async_copy(wc_hbm.at[r >> 4, pl.ds(r & 15, 1), :],
                             crows_v.at[pl.ds(k * 16 + j, 1), :], sem_c)
        return 0

    lax.fori_loop(0, _TGT_PER_W // 16, fire_t, 0)
    lax.fori_loop(0, _CTX_PER_W // 16, fire_c, 0)
    pltpu.make_async_copy(out_t_hbm.at[pl.ds(0, _TGT_PER_W)], trows_v,
                          sem_t).wait()
    pltpu.make_async_copy(out_c_hbm.at[pl.ds(0, _CTX_PER_W)], crows_v,
                          sem_c).wait()
    pltpu.sync_copy(trows_v, out_t_hbm.at[pl.ds(base_t, _TGT_PER_W)])
    pltpu.sync_copy(crows_v, out_c_hbm.at[pl.ds(base_c, _CTX_PER_W)])


_gather = pl.kernel(
    _gather_body,
    out_type=(
        jax.ShapeDtypeStruct((BATCH, EMBED), jnp.float32),
        jax.ShapeDtypeStruct((BATCH * CTX, EMBED), jnp.float32),
    ),
    mesh=plsc.VectorSubcoreMesh(core_axis_name="c", subcore_axis_name="s"),
    scratch_types=[
        pltpu.VMEM((_TGT_PER_W,), jnp.int32),
        pltpu.VMEM((_CTX_PER_W,), jnp.int32),
        pltpu.VMEM((_TGT_PER_W, EMBED), jnp.float32),
        pltpu.VMEM((_CTX_PER_W, EMBED), jnp.float32),
        pltpu.SemaphoreType.DMA,
        pltpu.SemaphoreType.DMA,
    ],
)


def _mm_body(a_ref, b_ref, o_ref):
    o_ref[...] = lax.dot_general(
        a_ref[...], b_ref[...],
        dimension_numbers=(((1,), (1,)), ((), ())),
        preferred_element_type=jnp.float32,
    )


_BN = 640


def kernel(target, context, W_target, W_context):
    tidx = jnp.asarray(target, jnp.int32).reshape(BATCH)
    cidx = jnp.asarray(context, jnp.int32).reshape(BATCH * CTX)
    wt3 = W_target.reshape(VOCAB // 16, 16, EMBED)
    wc3 = W_context.reshape(VOCAB // 16, 16, EMBED)
    emb_t, emb_c = _gather(wt3, wc3, tidx, cidx)
    out2d = pl.pallas_call(
        _mm_body,
        grid=(BATCH * CTX // _BN,),
        in_specs=[
            pl.BlockSpec((BATCH, EMBED), lambda j: (0, 0)),
            pl.BlockSpec((_BN, EMBED), lambda j: (j, 0)),
        ],
        out_specs=pl.BlockSpec((BATCH, _BN), lambda j: (0, j)),
        out_shape=jax.ShapeDtypeStruct((BATCH, BATCH * CTX), jnp.float32),
    )(emb_t, emb_c)
    return out2d.reshape(BATCH, BATCH, CTX)


# speedup vs baseline: 8.0376x; 3.5655x over previous
import jax
import jax.numpy as jnp
from jax import lax
from jax.experimental import pallas as pl
from jax.experimental.pallas import tpu as pltpu
from jax.experimental.pallas import tpu_sc as plsc

VOCAB = 1000000
EMBED = 64
BATCH = 1024
CTX = 5

_NC = 2
_NS = 16
_NW = _NC * _NS

_TILE = 128
_TGT_TILES = BATCH // _TILE
_CTX_TILES = BATCH * CTX // _TILE
_RING = 4

def _emit_job(tbl, idx_hbm, idx_base, out_hbm, col_base,
              idxbuf, staging, outbuf, sems):
    pltpu.sync_copy(idx_hbm.at[pl.ds(idx_base, _TILE)], idxbuf)
    row_idx = [lax.iota(jnp.int32, 16) + 16 * m for m in range(4)]

    def _select(slot, r, col):
        lane = jnp.full((16,), r & 127, jnp.int32)
        dst = jnp.full((16,), col, jnp.int32)
        for m in range(4):
            v = plsc.load_gather(staging[slot], [row_idx[m], lane])
            plsc.store_scatter(outbuf, [row_idx[m], dst], v)

    def chunk(k, vec_prev):
        vec = idxbuf[pl.ds(k * 16, 16)]
        for j in range(16):
            col = k * 16 + j
            slot = j % 4
            if j < 4:
                def drain(vp=vec_prev, s=slot, jj=j, c=col):
                    pltpu.make_async_copy(
                        tbl.at[:, pl.ds(0, _TILE)], staging[s],
                        sems[s]).wait()
                    _select(s, vp[12 + jj], c - 4)
                pl.when(k > 0)(drain)
            else:
                pltpu.make_async_copy(
                    tbl.at[:, pl.ds(0, _TILE)], staging[slot],
                    sems[slot]).wait()
                _select(slot, vec[j - 4], col - 4)
            blk = pl.multiple_of((vec[j] >> 7) * _TILE, _TILE)
            pltpu.async_copy(tbl.at[:, pl.ds(blk, _TILE)], staging[slot],
                             sems[slot])
        return vec

    vec_last = lax.fori_loop(0, _TILE // 16, chunk,
                             jnp.zeros((16,), jnp.int32))
    for j in range(4):
        pltpu.make_async_copy(tbl.at[:, pl.ds(0, _TILE)], staging[j],
                              sems[j]).wait()
        _select(j, vec_last[12 + j], _TILE - 4 + j)
    out_off = pl.multiple_of(col_base, _TILE)
    pltpu.sync_copy(outbuf, out_hbm.at[:, pl.ds(out_off, _TILE)])


def _gather_body(wtT_hbm, wcT_hbm, tidx_hbm, cidx_hbm, out_t_hbm, out_c_hbm,
                 idxbuf, s0, s1, s2, s3, outbuf, m0, m1, m2, m3):
    wid = lax.axis_index("s") * _NC + lax.axis_index("c")
    staging = [s0, s1, s2, s3]
    sems = [m0, m1, m2, m3]

    def job_t(tile):
        _emit_job(wtT_hbm, tidx_hbm, tile * _TILE, out_t_hbm, tile * _TILE,
                  idxbuf, staging, outbuf, sems)

    def job_c(tile):
        _emit_job(wcT_hbm, cidx_hbm, tile * _TILE, out_c_hbm, tile * _TILE,
                  idxbuf, staging, outbuf, sems)

    pl.when(wid < _TGT_TILES)(lambda: job_t(wid))
    pl.when(wid >= _TGT_TILES)(lambda: job_c(wid - _TGT_TILES))
    pl.when(wid < _NW // 2)(lambda: job_c(wid + _CTX_TILES - _NW // 2))


_gather = pl.kernel(
    _gather_body,
    out_type=(
        jax.ShapeDtypeStruct((EMBED, BATCH), jnp.float32),
        jax.ShapeDtypeStruct((EMBED, BATCH * CTX), jnp.float32),
    ),
    mesh=plsc.VectorSubcoreMesh(core_axis_name="c", subcore_axis_name="s"),
    compiler_params=pltpu.CompilerParams(needs_layout_passes=False),
    scratch_types=[
        pltpu.VMEM((_TILE,), jnp.int32),
        pltpu.VMEM((EMBED, _TILE), jnp.float32),
        pltpu.VMEM((EMBED, _TILE), jnp.float32),
        pltpu.VMEM((EMBED, _TILE), jnp.float32),
        pltpu.VMEM((EMBED, _TILE), jnp.float32),
        pltpu.VMEM((EMBED, _TILE), jnp.float32),
        pltpu.SemaphoreType.DMA,
        pltpu.SemaphoreType.DMA,
        pltpu.SemaphoreType.DMA,
        pltpu.SemaphoreType.DMA,
    ],
)


def _mm_body(a_ref, b_ref, o_ref):
    o_ref[0] = lax.dot_general(
        a_ref[...], b_ref[...],
        dimension_numbers=(((0,), (0,)), ((), ())),
        preferred_element_type=jnp.float32,
    )


_BN = 512


def kernel(target, context, W_target, W_context):
    tidx = jnp.asarray(target, jnp.int32).reshape(BATCH)
    cidx = jnp.asarray(context, jnp.int32).T.reshape(BATCH * CTX)
    embT_t, embT_c = _gather(W_target.T, W_context.T, tidx, cidx)
    out5 = pl.pallas_call(
        _mm_body,
        grid=(CTX, BATCH // _BN),
        in_specs=[
            pl.BlockSpec((EMBED, BATCH), lambda c, j: (0, 0)),
            pl.BlockSpec((EMBED, _BN),
                         lambda c, j: (0, c * (BATCH // _BN) + j)),
        ],
        out_specs=pl.BlockSpec((1, BATCH, _BN), lambda c, j: (c, 0, j)),
        out_shape=jax.ShapeDtypeStruct((CTX, BATCH, BATCH), jnp.float32),
    )(embT_t, embT_c)
    return jnp.transpose(out5, (1, 2, 0))
